# baseline (speedup 1.0000x reference)
import math

import jax
import jax.numpy as jnp
from jax.experimental import pallas as pl
from jax.experimental.pallas import tpu as pltpu

_N, _K, _D = 16384, 512, 16
_TILE_N = 2048


def _gmm_tile_kernel(xt_ref, must_ref, stdt_ref, w_ref, out_ref):
    xt = xt_ref[...]
    must = must_ref[...]
    stdt = stdt_ref[...]
    w = w_ref[...]

    log_std = jnp.log(stdt + 1e-12)
    iv = jnp.exp(-2.0 * log_std)

    m_w = jnp.max(w)
    lse_w = m_w + jnp.log(jnp.sum(jnp.exp(w - m_w)))
    cc = (-jnp.sum(log_std, axis=0, keepdims=True)
          + w
          - 0.5 * _D * math.log(2.0 * math.pi)
          - lse_w)
    c = cc - 0.5 * jnp.sum(must * must * iv, axis=0, keepdims=True)

    xa = jnp.concatenate([xt * xt, xt], axis=0)
    pa = jnp.concatenate([-0.5 * iv, must * iv], axis=0)

    def limbs(a):
        hi = a.astype(jnp.bfloat16)
        r = a - hi.astype(jnp.float32)
        mid = r.astype(jnp.bfloat16)
        lo = (r - mid.astype(jnp.float32)).astype(jnp.bfloat16)
        return hi, mid, lo

    ph, pm, plo = limbs(pa)
    xh, xm, xl = limbs(xa)
    pcat = jnp.concatenate([ph, ph, pm, ph, pm, plo], axis=0)
    xcat = jnp.concatenate([xh, xm, xh, xl, xm, xh], axis=0)
    logits = jax.lax.dot_general(
        pcat, xcat, (((0,), (0,)), ((), ())),
        preferred_element_type=jnp.float32) + c.reshape(_K, 1)

    idx = jnp.argmax(logits, axis=0)
    onehot = (jax.lax.broadcasted_iota(jnp.int32, (_K, _TILE_N), 0)
              == idx[None, :]).astype(jnp.bfloat16)

    muh, mum, mulo = limbs(must)
    ivh, ivm, ivl = limbs(iv)
    cch, ccm, ccl = limbs(cc)
    psel = jnp.concatenate(
        [muh, mum, mulo, ivh, ivm, ivl, cch, ccm, ccl], axis=0)
    sel = jnp.dot(psel, onehot, preferred_element_type=jnp.float32)
    mu_sel = sel[0:_D] + sel[_D:2 * _D] + sel[2 * _D:3 * _D]
    iv_sel = sel[3 * _D:4 * _D] + sel[4 * _D:5 * _D] + sel[5 * _D:6 * _D]
    cc_sel = sel[6 * _D:6 * _D + 1] + sel[6 * _D + 1:6 * _D + 2] \
        + sel[6 * _D + 2:6 * _D + 3]
    dq = xt - mu_sel
    exact = -0.5 * jnp.sum(dq * dq * iv_sel, axis=0, keepdims=True) + cc_sel

    logits2 = jnp.where(onehot > 0, exact, logits)
    m2 = jnp.max(logits2, axis=0, keepdims=True)
    lse = m2 + jnp.log(jnp.sum(jnp.exp(logits2 - m2), axis=0, keepdims=True))
    out_ref[...] = lse[None, :, :]


def kernel(x, mus, stdevs, weights):
    xt = x.T
    must = mus.T
    stdt = stdevs.T
    w2 = weights.reshape(1, _K)
    grid = (_N // _TILE_N,)
    out = pl.pallas_call(
        _gmm_tile_kernel,
        grid=grid,
        in_specs=[
            pl.BlockSpec((_D, _TILE_N), lambda i: (0, i)),
            pl.BlockSpec((_D, _K), lambda i: (0, 0)),
            pl.BlockSpec((_D, _K), lambda i: (0, 0)),
            pl.BlockSpec((1, _K), lambda i: (0, 0)),
        ],
        out_specs=pl.BlockSpec((1, 1, _TILE_N), lambda i: (i, 0, 0)),
        out_shape=jax.ShapeDtypeStruct((_N // _TILE_N, 1, _TILE_N), jnp.float32),
        compiler_params=pltpu.CompilerParams(
            dimension_semantics=("parallel",)),
    )(xt, must, stdt, w2)
    return out.reshape(_N)


# speedup vs baseline: 1.3207x; 1.3207x over previous
import math

import jax
import jax.numpy as jnp
from jax.experimental import pallas as pl
from jax.experimental.pallas import tpu as pltpu

_N, _K, _D = 16384, 512, 16
_TILE_N = 2048


def _gmm_tile_kernel(xt_ref, must_ref, stdt_ref, w_ref, out_ref):
    xt = xt_ref[...]
    must = must_ref[...]
    stdt = stdt_ref[...]
    w = w_ref[...]

    log_std = jnp.log(stdt + 1e-12)
    iv = jnp.exp(-2.0 * log_std)

    m_w = jnp.max(w)
    lse_w = m_w + jnp.log(jnp.sum(jnp.exp(w - m_w)))
    cc = (-jnp.sum(log_std, axis=0, keepdims=True)
          + w
          - 0.5 * _D * math.log(2.0 * math.pi)
          - lse_w)
    c = cc - 0.5 * jnp.sum(must * must * iv, axis=0, keepdims=True)

    xa = jnp.concatenate([xt * xt, xt], axis=0)
    pa = jnp.concatenate([-0.5 * iv, must * iv], axis=0)

    def limbs(a):
        hi = a.astype(jnp.bfloat16)
        r = a - hi.astype(jnp.float32)
        mid = r.astype(jnp.bfloat16)
        lo = (r - mid.astype(jnp.float32)).astype(jnp.bfloat16)
        return hi, mid, lo

    ph, pm, plo = limbs(pa)
    xh, xm, xl = limbs(xa)
    pcat = jnp.concatenate([ph, ph, pm, ph, pm, plo], axis=0)
    xcat = jnp.concatenate([xh, xm, xh, xl, xm, xh], axis=0)
    logits = jax.lax.dot_general(
        pcat, xcat, (((0,), (0,)), ((), ())),
        preferred_element_type=jnp.float32) + c.reshape(_K, 1)

    m = jnp.max(logits, axis=0, keepdims=True)
    onehot = (logits >= m).astype(jnp.bfloat16)
    cnt = jnp.sum(onehot.astype(jnp.float32), axis=0, keepdims=True)

    muh, mum, mulo = limbs(must)
    ivh, ivm, ivl = limbs(iv)
    cch, ccm, ccl = limbs(cc)
    psel = jnp.concatenate(
        [muh, mum, mulo, ivh, ivm, ivl, cch, ccm, ccl], axis=0)
    sel = jnp.dot(psel, onehot, preferred_element_type=jnp.float32)
    mu_sel = sel[0:_D] + sel[_D:2 * _D] + sel[2 * _D:3 * _D]
    iv_sel = sel[3 * _D:4 * _D] + sel[4 * _D:5 * _D] + sel[5 * _D:6 * _D]
    cc_sel = sel[6 * _D:6 * _D + 1] + sel[6 * _D + 1:6 * _D + 2] \
        + sel[6 * _D + 2:6 * _D + 3]
    dq = xt - mu_sel
    exact = -0.5 * jnp.sum(dq * dq * iv_sel, axis=0, keepdims=True) + cc_sel
    one = jnp.float32(1.0)
    exact = jnp.where(cnt <= one, exact, m)
    repl = jnp.where(cnt <= one, one, cnt)

    masked = logits - 1e30 * onehot.astype(jnp.float32)
    m_rest = jnp.max(masked, axis=0, keepdims=True)
    m2 = jnp.maximum(m_rest, exact)
    s_rest = jnp.sum(jnp.exp(masked - m2), axis=0, keepdims=True)
    lse = m2 + jnp.log(s_rest + repl * jnp.exp(exact - m2))
    out_ref[...] = lse[None, :, :]


def kernel(x, mus, stdevs, weights):
    xt = x.T
    must = mus.T
    stdt = stdevs.T
    w2 = weights.reshape(1, _K)
    grid = (_N // _TILE_N,)
    out = pl.pallas_call(
        _gmm_tile_kernel,
        grid=grid,
        in_specs=[
            pl.BlockSpec((_D, _TILE_N), lambda i: (0, i)),
            pl.BlockSpec((_D, _K), lambda i: (0, 0)),
            pl.BlockSpec((_D, _K), lambda i: (0, 0)),
            pl.BlockSpec((1, _K), lambda i: (0, 0)),
        ],
        out_specs=pl.BlockSpec((1, 1, _TILE_N), lambda i: (i, 0, 0)),
        out_shape=jax.ShapeDtypeStruct((_N // _TILE_N, 1, _TILE_N), jnp.float32),
        compiler_params=pltpu.CompilerParams(
            dimension_semantics=("parallel",)),
    )(xt, must, stdt, w2)
    return out.reshape(_N)


# speedup vs baseline: 1.3449x; 1.0184x over previous
import math

import jax
import jax.numpy as jnp
from jax.experimental import pallas as pl
from jax.experimental.pallas import tpu as pltpu

_N, _K, _D = 16384, 512, 16
_TILE_N = 2048


def _gmm_tile_kernel(xt_ref, must_ref, stdt_ref, w_ref, out_ref):
    xt = xt_ref[...]
    must = must_ref[...]
    stdt = stdt_ref[...]
    w = w_ref[...]

    log_std = jnp.log(stdt + 1e-12)
    iv = jnp.exp(-2.0 * log_std)

    m_w = jnp.max(w)
    lse_w = m_w + jnp.log(jnp.sum(jnp.exp(w - m_w)))
    cc = (-jnp.sum(log_std, axis=0, keepdims=True)
          + w
          - 0.5 * _D * math.log(2.0 * math.pi)
          - lse_w)
    c = cc - 0.5 * jnp.sum(must * must * iv, axis=0, keepdims=True)

    xa = jnp.concatenate([xt * xt, xt], axis=0)
    pa = jnp.concatenate([-0.5 * iv, must * iv], axis=0)

    def limbs(a):
        hi = a.astype(jnp.bfloat16)
        r = a - hi.astype(jnp.float32)
        mid = r.astype(jnp.bfloat16)
        lo = (r - mid.astype(jnp.float32)).astype(jnp.bfloat16)
        return hi, mid, lo

    ph, pm, plo = limbs(pa)
    xh, xm, xl = limbs(xa)
    pcat = jnp.concatenate([ph, ph, pm, ph, pm, plo], axis=0)
    xcat = jnp.concatenate([xh, xm, xh, xl, xm, xh], axis=0)
    logits = jax.lax.dot_general(
        pcat, xcat, (((0,), (0,)), ((), ())),
        preferred_element_type=jnp.float32) + c.reshape(_K, 1)

    m = jnp.max(logits, axis=0, keepdims=True)
    onehot = (logits >= m).astype(jnp.bfloat16)
    cnt = jnp.sum(onehot.astype(jnp.float32), axis=0, keepdims=True)

    muh, mum, mulo = limbs(must)
    ivh, ivm, ivl = limbs(iv)
    cch, ccm, ccl = limbs(cc)
    psel = jnp.concatenate(
        [muh, mum, mulo, ivh, ivm, ivl, cch, ccm, ccl], axis=0)
    sel = jnp.dot(psel, onehot, preferred_element_type=jnp.float32)
    mu_sel = sel[0:_D] + sel[_D:2 * _D] + sel[2 * _D:3 * _D]
    iv_sel = sel[3 * _D:4 * _D] + sel[4 * _D:5 * _D] + sel[5 * _D:6 * _D]
    cc_sel = sel[6 * _D:6 * _D + 1] + sel[6 * _D + 1:6 * _D + 2] \
        + sel[6 * _D + 2:6 * _D + 3]
    dq = xt - mu_sel
    exact = -0.5 * jnp.sum(dq * dq * iv_sel, axis=0, keepdims=True) + cc_sel
    one = jnp.float32(1.0)
    exact = jnp.where(cnt <= one, exact, m)
    repl = jnp.where(cnt <= one, one, cnt)

    masked = logits - 1e30 * onehot.astype(jnp.float32)
    m_rest = jnp.max(masked, axis=0, keepdims=True)
    m2 = jnp.maximum(m_rest, exact)
    s_rest = jnp.sum(jnp.exp(masked - m2), axis=0, keepdims=True)
    lse = m2 + jnp.log(s_rest + repl * jnp.exp(exact - m2))
    out_ref[...] = lse[None, :, :]


def kernel(x, mus, stdevs, weights):
    xt = x.T
    must = mus.T
    stdt = stdevs.T
    w2 = weights.reshape(1, _K)
    grid = (_N // _TILE_N,)
    out = pl.pallas_call(
        _gmm_tile_kernel,
        grid=grid,
        in_specs=[
            pl.BlockSpec((_D, _TILE_N), lambda i: (0, i)),
            pl.BlockSpec((_D, _K), lambda i: (0, 0)),
            pl.BlockSpec((_D, _K), lambda i: (0, 0)),
            pl.BlockSpec((1, _K), lambda i: (0, 0)),
        ],
        out_specs=pl.BlockSpec((1, 1, _TILE_N), lambda i: (i, 0, 0)),
        out_shape=jax.ShapeDtypeStruct((_N // _TILE_N, 1, _TILE_N), jnp.float32),
        compiler_params=pltpu.CompilerParams(
            dimension_semantics=("parallel",),
            allow_input_fusion=[True, True, True, True]),
    )(xt, must, stdt, w2)
    return out.reshape(_N)


# speedup vs baseline: 1.4457x; 1.0749x over previous
import math

import jax
import jax.numpy as jnp
from jax.experimental import pallas as pl
from jax.experimental.pallas import tpu as pltpu

_N, _K, _D = 16384, 512, 16
_TILE_N = 4096


def _gmm_tile_kernel(xt_ref, must_ref, stdt_ref, w_ref, out_ref):
    xt = xt_ref[...]
    must = must_ref[...]
    stdt = stdt_ref[...]
    w = w_ref[...]

    log_std = jnp.log(stdt + 1e-12)
    iv = jnp.exp(-2.0 * log_std)

    m_w = jnp.max(w)
    lse_w = m_w + jnp.log(jnp.sum(jnp.exp(w - m_w)))
    cc = (-jnp.sum(log_std, axis=0, keepdims=True)
          + w
          - 0.5 * _D * math.log(2.0 * math.pi)
          - lse_w)
    c = cc - 0.5 * jnp.sum(must * must * iv, axis=0, keepdims=True)

    xa = jnp.concatenate([xt * xt, xt], axis=0)
    pa = jnp.concatenate([-0.5 * iv, must * iv], axis=0)

    def limbs(a):
        hi = a.astype(jnp.bfloat16)
        r = a - hi.astype(jnp.float32)
        mid = r.astype(jnp.bfloat16)
        lo = (r - mid.astype(jnp.float32)).astype(jnp.bfloat16)
        return hi, mid, lo

    ph, pm, plo = limbs(pa)
    xh, xm, xl = limbs(xa)
    pcat = jnp.concatenate([ph, ph, pm, ph, pm, plo], axis=0)
    xcat = jnp.concatenate([xh, xm, xh, xl, xm, xh], axis=0)
    logits = jax.lax.dot_general(
        pcat, xcat, (((0,), (0,)), ((), ())),
        preferred_element_type=jnp.float32) + c.reshape(_K, 1)

    m = jnp.max(logits, axis=0, keepdims=True)
    onehot = (logits >= m).astype(jnp.bfloat16)
    cnt = jnp.sum(onehot.astype(jnp.float32), axis=0, keepdims=True)

    muh, mum, mulo = limbs(must)
    ivh, ivm, ivl = limbs(iv)
    cch, ccm, ccl = limbs(cc)
    psel = jnp.concatenate(
        [muh, mum, mulo, ivh, ivm, ivl, cch, ccm, ccl], axis=0)
    sel = jnp.dot(psel, onehot, preferred_element_type=jnp.float32)
    mu_sel = sel[0:_D] + sel[_D:2 * _D] + sel[2 * _D:3 * _D]
    iv_sel = sel[3 * _D:4 * _D] + sel[4 * _D:5 * _D] + sel[5 * _D:6 * _D]
    cc_sel = sel[6 * _D:6 * _D + 1] + sel[6 * _D + 1:6 * _D + 2] \
        + sel[6 * _D + 2:6 * _D + 3]
    dq = xt - mu_sel
    exact = -0.5 * jnp.sum(dq * dq * iv_sel, axis=0, keepdims=True) + cc_sel
    one = jnp.float32(1.0)
    exact = jnp.where(cnt <= one, exact, m)
    repl = jnp.where(cnt <= one, one, cnt)

    masked = logits - 1e30 * onehot.astype(jnp.float32)
    m_rest = jnp.max(masked, axis=0, keepdims=True)
    m2 = jnp.maximum(m_rest, exact)
    s_rest = jnp.sum(jnp.exp(masked - m2), axis=0, keepdims=True)
    lse = m2 + jnp.log(s_rest + repl * jnp.exp(exact - m2))
    out_ref[...] = lse[None, :, :]


def kernel(x, mus, stdevs, weights):
    xt = x.T
    must = mus.T
    stdt = stdevs.T
    w2 = weights.reshape(1, _K)
    grid = (_N // _TILE_N,)
    out = pl.pallas_call(
        _gmm_tile_kernel,
        grid=grid,
        in_specs=[
            pl.BlockSpec((_D, _TILE_N), lambda i: (0, i)),
            pl.BlockSpec((_D, _K), lambda i: (0, 0)),
            pl.BlockSpec((_D, _K), lambda i: (0, 0)),
            pl.BlockSpec((1, _K), lambda i: (0, 0)),
        ],
        out_specs=pl.BlockSpec((1, 1, _TILE_N), lambda i: (i, 0, 0)),
        out_shape=jax.ShapeDtypeStruct((_N // _TILE_N, 1, _TILE_N), jnp.float32),
        compiler_params=pltpu.CompilerParams(
            dimension_semantics=("parallel",),
            allow_input_fusion=[True, True, True, True]),
    )(xt, must, stdt, w2)
    return out.reshape(_N)


# speedup vs baseline: 1.5213x; 1.0523x over previous
import math

import jax
import jax.numpy as jnp
from jax.experimental import pallas as pl
from jax.experimental.pallas import tpu as pltpu

_N, _K, _D = 16384, 512, 16
_TILE_N = 8192


def _gmm_tile_kernel(xt_ref, must_ref, stdt_ref, w_ref, out_ref):
    xt = xt_ref[...]
    must = must_ref[...]
    stdt = stdt_ref[...]
    w = w_ref[...]

    log_std = jnp.log(stdt + 1e-12)
    iv = jnp.exp(-2.0 * log_std)

    m_w = jnp.max(w)
    lse_w = m_w + jnp.log(jnp.sum(jnp.exp(w - m_w)))
    cc = (-jnp.sum(log_std, axis=0, keepdims=True)
          + w
          - 0.5 * _D * math.log(2.0 * math.pi)
          - lse_w)
    c = cc - 0.5 * jnp.sum(must * must * iv, axis=0, keepdims=True)

    xa = jnp.concatenate([xt * xt, xt], axis=0)
    pa = jnp.concatenate([-0.5 * iv, must * iv], axis=0)

    def limbs(a):
        hi = a.astype(jnp.bfloat16)
        r = a - hi.astype(jnp.float32)
        mid = r.astype(jnp.bfloat16)
        lo = (r - mid.astype(jnp.float32)).astype(jnp.bfloat16)
        return hi, mid, lo

    ph, pm, plo = limbs(pa)
    xh, xm, xl = limbs(xa)
    pcat = jnp.concatenate([ph, ph, pm, ph, pm, plo], axis=0)
    xcat = jnp.concatenate([xh, xm, xh, xl, xm, xh], axis=0)
    logits = jax.lax.dot_general(
        pcat, xcat, (((0,), (0,)), ((), ())),
        preferred_element_type=jnp.float32) + c.reshape(_K, 1)

    m = jnp.max(logits, axis=0, keepdims=True)
    onehot = (logits >= m).astype(jnp.bfloat16)
    cnt = jnp.sum(onehot.astype(jnp.float32), axis=0, keepdims=True)

    muh, mum, mulo = limbs(must)
    ivh, ivm, ivl = limbs(iv)
    cch, ccm, ccl = limbs(cc)
    psel = jnp.concatenate(
        [muh, mum, mulo, ivh, ivm, ivl, cch, ccm, ccl], axis=0)
    sel = jnp.dot(psel, onehot, preferred_element_type=jnp.float32)
    mu_sel = sel[0:_D] + sel[_D:2 * _D] + sel[2 * _D:3 * _D]
    iv_sel = sel[3 * _D:4 * _D] + sel[4 * _D:5 * _D] + sel[5 * _D:6 * _D]
    cc_sel = sel[6 * _D:6 * _D + 1] + sel[6 * _D + 1:6 * _D + 2] \
        + sel[6 * _D + 2:6 * _D + 3]
    dq = xt - mu_sel
    exact = -0.5 * jnp.sum(dq * dq * iv_sel, axis=0, keepdims=True) + cc_sel
    one = jnp.float32(1.0)
    exact = jnp.where(cnt <= one, exact, m)
    repl = jnp.where(cnt <= one, one, cnt)

    masked = logits - 1e30 * onehot.astype(jnp.float32)
    m_rest = jnp.max(masked, axis=0, keepdims=True)
    m2 = jnp.maximum(m_rest, exact)
    s_rest = jnp.sum(jnp.exp(masked - m2), axis=0, keepdims=True)
    lse = m2 + jnp.log(s_rest + repl * jnp.exp(exact - m2))
    out_ref[...] = lse[None, :, :]


def kernel(x, mus, stdevs, weights):
    xt = x.T
    must = mus.T
    stdt = stdevs.T
    w2 = weights.reshape(1, _K)
    grid = (_N // _TILE_N,)
    out = pl.pallas_call(
        _gmm_tile_kernel,
        grid=grid,
        in_specs=[
            pl.BlockSpec((_D, _TILE_N), lambda i: (0, i)),
            pl.BlockSpec((_D, _K), lambda i: (0, 0)),
            pl.BlockSpec((_D, _K), lambda i: (0, 0)),
            pl.BlockSpec((1, _K), lambda i: (0, 0)),
        ],
        out_specs=pl.BlockSpec((1, 1, _TILE_N), lambda i: (i, 0, 0)),
        out_shape=jax.ShapeDtypeStruct((_N // _TILE_N, 1, _TILE_N), jnp.float32),
        compiler_params=pltpu.CompilerParams(
            dimension_semantics=("parallel",),
            allow_input_fusion=[True, True, True, True]),
    )(xt, must, stdt, w2)
    return out.reshape(_N)
